# baseline (speedup 1.0000x reference)
import functools

import jax
import jax.numpy as jnp
from jax import lax
from jax.experimental import pallas as pl
from jax.experimental.pallas import tpu as pltpu
from jax.experimental.pallas import tpu_sc as plsc
import jax.experimental.layout
from jax._src.layout import AutoLayout as _AUTO

_D = 512
_G = 4096
_GW = 20

_info = plsc.get_sparse_core_info()
_NC, _NS = _info.num_cores, _info.num_subcores
_NW = _NC * _NS
_GPW = _G // _NW
_CPG = 2
_RPC = _CPG * _GW
_NCHUNK = _GPW // _CPG
_NL = _D // 16


def _make_gather():
    mesh = plsc.VectorSubcoreMesh(core_axis_name="c", subcore_axis_name="s")

    @functools.partial(
        pl.kernel,
        mesh=mesh,
        out_type=jax.ShapeDtypeStruct((_G, _GW, _D), jnp.float32),
        scratch_types=[
            pltpu.VMEM((_NCHUNK, _RPC), jnp.int32),
            pltpu.VMEM((_RPC, _D), jnp.float32),
            pltpu.VMEM((_RPC, _D), jnp.float32),
            pltpu.VMEM((_GW, _D), jnp.float32),
            pltpu.VMEM((_GW, _D), jnp.float32),
            pltpu.VMEM((_GW, _D), jnp.float32),
            pltpu.VMEM((_GW, _D), jnp.float32),
            pltpu.SemaphoreType.DMA,
            pltpu.SemaphoreType.DMA,
            pltpu.SemaphoreType.DMA,
            pltpu.SemaphoreType.DMA,
        ],
    )
    def gather_k(idx_hbm, table_hbm, out_hbm, idx_v,
                 buf0, buf1, st00, st01, st10, st11, g0, g1, w0, w1):
        buf = [buf0, buf1]
        stage = [[st00, st01], [st10, st11]]
        gsem = [g0, g1]
        wsem = [w0, w1]

        wid = lax.axis_index("s") * _NC + lax.axis_index("c")
        gbase = wid * _GPW
        pltpu.sync_copy(idx_hbm.at[pl.ds(wid * _NCHUNK, _NCHUNK)], idx_v)

        def start_gather(c, b):
            pltpu.async_copy(table_hbm.at[idx_v.at[c]], buf[b], gsem[b])

        def wait_gather(c, b):
            pltpu.make_async_copy(table_hbm.at[idx_v.at[c]], buf[b],
                                  gsem[b]).wait()

        def repack(b):
            def rrow(r, carry):
                for k in range(_CPG):
                    for l in range(_NL):
                        stage[b][k][r, pl.ds(l * 16, 16)] = (
                            buf[b][k * _GW + r, pl.ds(l * 16, 16)])
                return carry
            lax.fori_loop(0, _GW, rrow, 0)

        def start_wb(c, b):
            for k in range(_CPG):
                pltpu.async_copy(stage[b][k],
                                 out_hbm.at[gbase + c * _CPG + k], wsem[b])

        def wait_wb(c, b):
            for k in range(_CPG):
                pltpu.make_async_copy(stage[b][k],
                                      out_hbm.at[gbase + c * _CPG + k],
                                      wsem[b]).wait()

        def step(c, b, first, last):
            wait_gather(c, b)
            if not first:
                wait_wb(c - 1, 1 - b)
            if not last:
                start_gather(c + 1, 1 - b)
            repack(b)
            start_wb(c, b)

        start_gather(0, 0)
        step(0, 0, True, False)
        step(1, 1, False, False)

        def round_body(p, carry):
            step(2 * p, 0, False, False)
            step(2 * p + 1, 1, False, False)
            return carry

        lax.fori_loop(1, _NCHUNK // 2 - 1, round_body, 0)

        step(_NCHUNK - 2, 0, False, False)
        step(_NCHUNK - 1, 1, False, True)
        wait_wb(_NCHUNK - 1, 1)

    return gather_k


_gather = _make_gather()


def _kernel_impl(indices, weight):
    idx = indices.astype(jnp.int32).reshape(_NW * _NCHUNK, _RPC)
    return _gather(idx, weight)


kernel = jax.jit(_kernel_impl,
                 out_shardings=jax.experimental.layout.Format(_AUTO))


# speedup vs baseline: 1.3725x; 1.3725x over previous
import functools

import jax
import jax.numpy as jnp
from jax import lax
from jax.experimental import pallas as pl
from jax.experimental.pallas import tpu as pltpu
from jax.experimental.pallas import tpu_sc as plsc

_D = 512
_G = 4096
_GW = 20
_GP = 24

_info = plsc.get_sparse_core_info()
_NC, _NS = _info.num_cores, _info.num_subcores
_NW = _NC * _NS
_GPW = _G // _NW
_CPG = 2
_RPC = _CPG * _GP
_NCHUNK = _GPW // _CPG
_NB = 4
_LOOK = 2
_NROUND = _NCHUNK // _NB


def _make_gather():
    mesh = plsc.VectorSubcoreMesh(core_axis_name="c", subcore_axis_name="s")

    @functools.partial(
        pl.kernel,
        mesh=mesh,
        out_type=jax.ShapeDtypeStruct((_G * _GP, _D), jnp.float32),
        scratch_types=[
            pltpu.VMEM((_NCHUNK, _RPC), jnp.int32),
            pltpu.VMEM((_RPC, _D), jnp.float32),
            pltpu.VMEM((_RPC, _D), jnp.float32),
            pltpu.VMEM((_RPC, _D), jnp.float32),
            pltpu.VMEM((_RPC, _D), jnp.float32),
            pltpu.SemaphoreType.DMA,
            pltpu.SemaphoreType.DMA,
            pltpu.SemaphoreType.DMA,
            pltpu.SemaphoreType.DMA,
            pltpu.SemaphoreType.DMA,
            pltpu.SemaphoreType.DMA,
            pltpu.SemaphoreType.DMA,
            pltpu.SemaphoreType.DMA,
        ],
    )
    def gather_k(idx_hbm, table_hbm, out_hbm, idx_v,
                 b0, b1, b2, b3, g0, g1, g2, g3, w0, w1, w2, w3):
        buf = [b0, b1, b2, b3]
        gsem = [g0, g1, g2, g3]
        wsem = [w0, w1, w2, w3]

        wid = lax.axis_index("s") * _NC + lax.axis_index("c")
        rbase = wid * _NCHUNK * _RPC
        pltpu.sync_copy(idx_hbm.at[pl.ds(wid * _NCHUNK, _NCHUNK)], idx_v)

        def start_gather(c, b):
            pltpu.async_copy(table_hbm.at[idx_v.at[c]], buf[b], gsem[b])

        def wait_gather(c, b):
            pltpu.make_async_copy(table_hbm.at[idx_v.at[c]], buf[b],
                                  gsem[b]).wait()

        def start_wb(c, b):
            pltpu.async_copy(buf[b], out_hbm.at[pl.ds(rbase + c * _RPC, _RPC)],
                             wsem[b])

        def wait_wb(c, b):
            pltpu.make_async_copy(buf[b],
                                  out_hbm.at[pl.ds(rbase + c * _RPC, _RPC)],
                                  wsem[b]).wait()

        start_gather(0, 0)
        start_gather(1, 1)

        for b in range(_NB):
            wait_gather(b, b)
            start_wb(b, b)
            cn = b + _LOOK
            bn = cn % _NB
            if b >= _LOOK:
                wait_wb(cn - _NB, bn)
            start_gather(cn, bn)

        def round_body(p, carry):
            for b in range(_NB):
                c = _NB * p + b
                cn = c + _LOOK
                bn = (b + _LOOK) % _NB
                wait_gather(c, b)
                start_wb(c, b)
                wait_wb(cn - _NB, bn)
                start_gather(cn, bn)
            return carry

        lax.fori_loop(1, _NROUND - 1, round_body, 0)

        for b in range(_NB):
            c = _NB * (_NROUND - 1) + b
            cn = c + _LOOK
            bn = (b + _LOOK) % _NB
            wait_gather(c, b)
            start_wb(c, b)
            if cn < _NCHUNK:
                wait_wb(cn - _NB, bn)
                start_gather(cn, bn)

        for b in range(_NB):
            wait_wb(_NCHUNK - _NB + b, b)

    return gather_k


_gather = _make_gather()


@jax.jit
def kernel(indices, weight):
    idx = indices.astype(jnp.int32)
    idx24 = jnp.concatenate([idx, idx[:, _GW - (_GP - _GW):]], axis=1)
    idx_chunks = idx24.reshape(_NW * _NCHUNK, _RPC)
    out = _gather(idx_chunks, weight)
    return out.reshape(_G, _GP, _D)[:, :_GW, :]
